# baseline (speedup 1.0000x reference)
import jax
import jax.numpy as jnp
from jax import lax
from jax.experimental import pallas as pl
from jax.experimental.pallas import tpu as pltpu
from jax.experimental.pallas import tpu_sc as plsc

N = 10000
E = 320000
D = 128
H = D // 2

NS = 16
NP = 10112
EPT = E // NS
CHUNK = 80
NCHUNK = EPT // CHUNK
ROWS_PT = NP // NS


def _mm_body(h_ref, norm_ref, w_ref, b_ref, ah_ref, bd_ref, eh_ref):
    hh = h_ref[...] * norm_ref[...]
    p = jnp.dot(hh, w_ref[...], preferred_element_type=jnp.float32) + b_ref[...]
    ah_ref[...] = p[:, 0:128]
    b_part = p[:, 128:256]
    d_part = jnp.exp(-p[:, 256:384])
    e_part = jnp.exp(-p[:, 384:512])
    bd_ref[0] = jnp.concatenate([b_part[:, :H], d_part[:, :H]], axis=1)
    bd_ref[1] = jnp.concatenate([b_part[:, H:], d_part[:, H:]], axis=1)
    eh_ref[0] = e_part
    eh_ref[1] = jnp.concatenate([e_part[:, H:], e_part[:, :H]], axis=1)


def _matmuls(h, norm, wcat, bcat):
    bn = 1000
    nb = N // bn
    return pl.pallas_call(
        _mm_body,
        grid=(nb,),
        in_specs=[
            pl.BlockSpec((bn, D), lambda i: (i, 0)),
            pl.BlockSpec((bn, 1), lambda i: (i, 0)),
            pl.BlockSpec((D, 4 * D), lambda i: (0, 0)),
            pl.BlockSpec((1, 4 * D), lambda i: (0, 0)),
        ],
        out_specs=[
            pl.BlockSpec((bn, D), lambda i: (i, 0)),
            pl.BlockSpec((2, bn, D), lambda i: (0, i, 0)),
            pl.BlockSpec((2, bn, D), lambda i: (0, i, 0)),
        ],
        out_shape=[
            jax.ShapeDtypeStruct((N, D), jnp.float32),
            jax.ShapeDtypeStruct((2, N, D), jnp.float32),
            jax.ShapeDtypeStruct((2, N, D), jnp.float32),
        ],
    )(h, norm, wcat, bcat)


IDXB = 2000
CPB = IDXB // CHUNK


def _edge_body(bd_hbm, eh_hbm, src_hbm, dst_hbm, zeros_hbm, out_hbm,
               big_src, big_dst, idx_srcc, idx_dstc, idx_dsc, bd_v, eh_v,
               acc, sem0, sem1):
    c = lax.axis_index("c")
    s = lax.axis_index("s")
    c_n = c * N
    sems = (sem0, sem1)

    row0 = s * ROWS_PT
    pltpu.sync_copy(zeros_hbm.at[pl.ds(row0, ROWS_PT)],
                    acc.at[pl.ds(row0, ROWS_PT)])
    plsc.subcore_barrier()

    base = s * EPT

    def load_batch(g):
        off = base + g * IDXB
        pltpu.sync_copy(src_hbm.at[pl.ds(off, IDXB)], big_src)
        pltpu.sync_copy(dst_hbm.at[pl.ds(off, IDXB)], big_dst)

    def prep_idx(b, k):
        lo = lax.rem(k, CPB) * CHUNK
        for j in range(CHUNK // 16):
            sl_in = pl.ds(lo + j * 16, 16)
            sl = pl.ds(j * 16, 16)
            idx_srcc.at[b][sl] = big_src[sl_in] + c_n
            dv = big_dst[sl_in]
            idx_dstc.at[b][sl] = dv + c_n
            idx_dsc.at[b][sl] = dv

    def start_gathers(b):
        pltpu.async_copy(bd_hbm.at[idx_srcc.at[b]], bd_v.at[b], sems[b])
        pltpu.async_copy(eh_hbm.at[idx_dstc.at[b]], eh_v.at[b], sems[b])

    def wait_gathers(b):
        pltpu.make_async_copy(bd_hbm.at[idx_srcc.at[b]], bd_v.at[b],
                              sems[b]).wait()
        pltpu.make_async_copy(eh_hbm.at[idx_dstc.at[b]], eh_v.at[b],
                              sems[b]).wait()

    load_batch(0)
    for b in range(2):
        prep_idx(b, b)
        start_gathers(b)

    def pair(p, carry):
        for b in range(2):
            k = 2 * p + b
            wait_gathers(b)

            def edge(i, carry2):
                for j in range(H // 16):
                    slj = pl.ds(j * 16, 16)
                    slj2 = pl.ds(H + j * 16, 16)
                    bh = bd_v.at[b][i, slj]
                    ed = bd_v.at[b][i, slj2]
                    ee = eh_v.at[b][i, slj]
                    sg = 1.0 / (1.0 + ed * ee)
                    bd_v.at[b][i, slj] = sg * bh
                    bd_v.at[b][i, slj2] = sg
                return carry2

            lax.fori_loop(0, CHUNK, edge, 0)
            pltpu.sync_copy(bd_v.at[b], acc.at[idx_dsc.at[b]], add=True)

            @pl.when(k < NCHUNK - 2)
            def _ahead():
                @pl.when(lax.rem(k + 2, CPB) == 0)
                def _refill():
                    load_batch((k + 2) // CPB)

                prep_idx(b, k + 2)
                start_gathers(b)

        return carry

    lax.fori_loop(0, NCHUNK // 2, pair, 0)
    plsc.subcore_barrier()
    pltpu.sync_copy(acc.at[pl.ds(row0, ROWS_PT)],
                    out_hbm.at[pl.ds(c * NP + row0, ROWS_PT)])


def _edge_phase(bd, eh, src, dst, zeros):
    mesh = plsc.VectorSubcoreMesh(core_axis_name="c", subcore_axis_name="s")
    k = pl.kernel(
        _edge_body,
        out_type=jax.ShapeDtypeStruct((2 * NP, D), jnp.float32),
        mesh=mesh,
        scratch_types=[
            pltpu.VMEM((IDXB,), jnp.int32),
            pltpu.VMEM((IDXB,), jnp.int32),
            pltpu.VMEM((2, CHUNK), jnp.int32),
            pltpu.VMEM((2, CHUNK), jnp.int32),
            pltpu.VMEM((2, CHUNK), jnp.int32),
            pltpu.VMEM((2, CHUNK, D), jnp.float32),
            pltpu.VMEM((2, CHUNK, D), jnp.float32),
            pltpu.VMEM_SHARED((NP, D), jnp.float32),
            pltpu.SemaphoreType.DMA,
            pltpu.SemaphoreType.DMA,
        ],
    )
    return k(bd, eh, src, dst, zeros)


def _h2_body(ah_ref, a0_ref, a1_ref, norm_ref, h2_ref, sum_ref, ssq_ref):
    i = pl.program_id(0)
    num = jnp.concatenate([a0_ref[:, :H], a1_ref[:, :H]], axis=1)
    den = jnp.concatenate([a0_ref[:, H:], a1_ref[:, H:]], axis=1)
    h2 = (ah_ref[...] + num / (den + 1e-6)) * norm_ref[...]
    h2_ref[...] = h2

    @pl.when(i == 0)
    def _init():
        sum_ref[...] = jnp.zeros_like(sum_ref)
        ssq_ref[...] = jnp.zeros_like(ssq_ref)

    sum_ref[...] += jnp.sum(h2, axis=0, keepdims=True)
    ssq_ref[...] += jnp.sum(h2 * h2, axis=0, keepdims=True)


def _h2_stats(ah, acc0, acc1, norm):
    bn = 1000
    nb = N // bn
    return pl.pallas_call(
        _h2_body,
        grid=(nb,),
        in_specs=[
            pl.BlockSpec((bn, D), lambda i: (i, 0)),
            pl.BlockSpec((bn, D), lambda i: (i, 0)),
            pl.BlockSpec((bn, D), lambda i: (i, 0)),
            pl.BlockSpec((bn, 1), lambda i: (i, 0)),
        ],
        out_specs=[
            pl.BlockSpec((bn, D), lambda i: (i, 0)),
            pl.BlockSpec((1, D), lambda i: (0, 0)),
            pl.BlockSpec((1, D), lambda i: (0, 0)),
        ],
        out_shape=[
            jax.ShapeDtypeStruct((N, D), jnp.float32),
            jax.ShapeDtypeStruct((1, D), jnp.float32),
            jax.ShapeDtypeStruct((1, D), jnp.float32),
        ],
    )(ah, acc0, acc1, norm)


def _bn_body(h_ref, h2_ref, sum_ref, ssq_ref, g_ref, b_ref, out_ref):
    mean = sum_ref[...] / N
    var = ssq_ref[...] / N - mean * mean
    inv = lax.rsqrt(var + 1e-5)
    out_ref[...] = h_ref[...] + (h2_ref[...] - mean) * inv * g_ref[...] + b_ref[...]


def _bn_apply(h, h2, ssum, ssq, gamma, beta):
    bn = 1000
    nb = N // bn
    return pl.pallas_call(
        _bn_body,
        grid=(nb,),
        in_specs=[
            pl.BlockSpec((bn, D), lambda i: (i, 0)),
            pl.BlockSpec((bn, D), lambda i: (i, 0)),
            pl.BlockSpec((1, D), lambda i: (0, 0)),
            pl.BlockSpec((1, D), lambda i: (0, 0)),
            pl.BlockSpec((1, D), lambda i: (0, 0)),
            pl.BlockSpec((1, D), lambda i: (0, 0)),
        ],
        out_specs=pl.BlockSpec((bn, D), lambda i: (i, 0)),
        out_shape=jax.ShapeDtypeStruct((N, D), jnp.float32),
    )(h, h2, ssum, ssq, gamma, beta)


def kernel(h, edge_index, e, norm, WA, bA, WB, bB, WD, bD, WE, bE, gamma, beta):
    wcat = jnp.concatenate([WA, WB, WD, WE], axis=1)
    bcat = jnp.concatenate([bA, bB, bD, bE])[None, :]
    ah, bd, eh = _matmuls(h, norm, wcat, bcat)
    bd = bd.reshape(2 * N, D)
    eh = eh.reshape(2 * N, D)
    src = edge_index[0]
    dst = edge_index[1]
    zeros = jnp.zeros((NP, D), jnp.float32)
    acc = _edge_phase(bd, eh, src, dst, zeros)
    h2, ssum, ssq = _h2_stats(ah, acc[:N], acc[NP:NP + N], norm)
    out = _bn_apply(h, h2, ssum, ssq, gamma[None, :], beta[None, :])
    return (out, e)


# speedup vs baseline: 1.0060x; 1.0060x over previous
import jax
import jax.numpy as jnp
from jax import lax
from jax.experimental import pallas as pl
from jax.experimental.pallas import tpu as pltpu
from jax.experimental.pallas import tpu_sc as plsc

N = 10000
E = 320000
D = 128
H = D // 2

NS = 16
NP = 10112
EPT = E // NS
CHUNK = 80
NCHUNK = EPT // CHUNK
ROWS_PT = NP // NS


def _mm_body(h_ref, norm_ref, w_ref, b_ref, ah_ref, bd_ref, eh_ref):
    hh = h_ref[...] * norm_ref[...]
    p = jnp.dot(hh, w_ref[...], preferred_element_type=jnp.float32) + b_ref[...]
    ah_ref[...] = p[:, 0:128]
    b_part = p[:, 128:256]
    d_part = jnp.exp(-p[:, 256:384])
    e_part = jnp.exp(-p[:, 384:512])
    bd_ref[0] = jnp.concatenate([b_part[:, :H], d_part[:, :H]], axis=1)
    bd_ref[1] = jnp.concatenate([b_part[:, H:], d_part[:, H:]], axis=1)
    eh_ref[0] = e_part
    eh_ref[1] = jnp.concatenate([e_part[:, H:], e_part[:, :H]], axis=1)


def _matmuls(h, norm, wcat, bcat):
    bn = 1000
    nb = N // bn
    return pl.pallas_call(
        _mm_body,
        grid=(nb,),
        in_specs=[
            pl.BlockSpec((bn, D), lambda i: (i, 0)),
            pl.BlockSpec((bn, 1), lambda i: (i, 0)),
            pl.BlockSpec((D, 4 * D), lambda i: (0, 0)),
            pl.BlockSpec((1, 4 * D), lambda i: (0, 0)),
        ],
        out_specs=[
            pl.BlockSpec((bn, D), lambda i: (i, 0)),
            pl.BlockSpec((2, bn, D), lambda i: (0, i, 0)),
            pl.BlockSpec((2, bn, D), lambda i: (0, i, 0)),
        ],
        out_shape=[
            jax.ShapeDtypeStruct((N, D), jnp.float32),
            jax.ShapeDtypeStruct((2, N, D), jnp.float32),
            jax.ShapeDtypeStruct((2, N, D), jnp.float32),
        ],
    )(h, norm, wcat, bcat)


IDXB = 2000
CPB = IDXB // CHUNK


def _edge_body(bd_hbm, eh_hbm, src_hbm, dst_hbm, out_hbm,
               big_src, big_dst, idx_srcc, idx_dstc, idx_dsc, bd_v, eh_v,
               acc, sem0, sem1):
    c = lax.axis_index("c")
    s = lax.axis_index("s")
    c_n = c * N
    sems = (sem0, sem1)

    row0 = s * ROWS_PT
    zero16 = jnp.zeros((16,), jnp.float32)

    def zrow(i, carry0):
        for j in range(D // 16):
            bd_v.at[0][i, pl.ds(j * 16, 16)] = zero16
        return carry0

    lax.fori_loop(0, CHUNK, zrow, 0)
    for m in range(ROWS_PT // CHUNK):
        pltpu.sync_copy(bd_v.at[0], acc.at[pl.ds(row0 + m * CHUNK, CHUNK)])
    rem = ROWS_PT % CHUNK
    if rem:
        pltpu.sync_copy(
            bd_v.at[0].at[pl.ds(0, rem)],
            acc.at[pl.ds(row0 + ROWS_PT - rem, rem)])
    plsc.subcore_barrier()

    base = s * EPT

    def load_batch(g):
        off = base + g * IDXB
        pltpu.sync_copy(src_hbm.at[pl.ds(off, IDXB)], big_src)
        pltpu.sync_copy(dst_hbm.at[pl.ds(off, IDXB)], big_dst)

    def prep_idx(b, k):
        lo = lax.rem(k, CPB) * CHUNK
        for j in range(CHUNK // 16):
            sl_in = pl.ds(lo + j * 16, 16)
            sl = pl.ds(j * 16, 16)
            idx_srcc.at[b][sl] = big_src[sl_in] + c_n
            dv = big_dst[sl_in]
            idx_dstc.at[b][sl] = dv + c_n
            idx_dsc.at[b][sl] = dv

    def start_gathers(b):
        pltpu.async_copy(bd_hbm.at[idx_srcc.at[b]], bd_v.at[b], sems[b])
        pltpu.async_copy(eh_hbm.at[idx_dstc.at[b]], eh_v.at[b], sems[b])

    def wait_gathers(b):
        pltpu.make_async_copy(bd_hbm.at[idx_srcc.at[b]], bd_v.at[b],
                              sems[b]).wait()
        pltpu.make_async_copy(eh_hbm.at[idx_dstc.at[b]], eh_v.at[b],
                              sems[b]).wait()

    load_batch(0)
    for b in range(2):
        prep_idx(b, b)
        start_gathers(b)

    def pair(p, carry):
        for b in range(2):
            k = 2 * p + b
            wait_gathers(b)

            def edge(i, carry2):
                for j in range(H // 16):
                    slj = pl.ds(j * 16, 16)
                    slj2 = pl.ds(H + j * 16, 16)
                    bh = bd_v.at[b][i, slj]
                    ed = bd_v.at[b][i, slj2]
                    ee = eh_v.at[b][i, slj]
                    sg = 1.0 / (1.0 + ed * ee)
                    bd_v.at[b][i, slj] = sg * bh
                    bd_v.at[b][i, slj2] = sg
                return carry2

            lax.fori_loop(0, CHUNK, edge, 0)
            pltpu.sync_copy(bd_v.at[b], acc.at[idx_dsc.at[b]], add=True)

            @pl.when(k < NCHUNK - 2)
            def _ahead():
                @pl.when(lax.rem(k + 2, CPB) == 0)
                def _refill():
                    load_batch((k + 2) // CPB)

                prep_idx(b, k + 2)
                start_gathers(b)

        return carry

    lax.fori_loop(0, NCHUNK // 2, pair, 0)
    plsc.subcore_barrier()
    pltpu.sync_copy(acc.at[pl.ds(row0, ROWS_PT)],
                    out_hbm.at[pl.ds(c * NP + row0, ROWS_PT)])


def _edge_phase(bd, eh, src, dst):
    mesh = plsc.VectorSubcoreMesh(core_axis_name="c", subcore_axis_name="s")
    k = pl.kernel(
        _edge_body,
        out_type=jax.ShapeDtypeStruct((2 * NP, D), jnp.float32),
        mesh=mesh,
        scratch_types=[
            pltpu.VMEM((IDXB,), jnp.int32),
            pltpu.VMEM((IDXB,), jnp.int32),
            pltpu.VMEM((2, CHUNK), jnp.int32),
            pltpu.VMEM((2, CHUNK), jnp.int32),
            pltpu.VMEM((2, CHUNK), jnp.int32),
            pltpu.VMEM((2, CHUNK, D), jnp.float32),
            pltpu.VMEM((2, CHUNK, D), jnp.float32),
            pltpu.VMEM_SHARED((NP, D), jnp.float32),
            pltpu.SemaphoreType.DMA,
            pltpu.SemaphoreType.DMA,
        ],
    )
    return k(bd, eh, src, dst)


BN_BLK = 1000
BN_NB = N // BN_BLK


def _fin_body(ah_ref, a0_ref, a1_ref, norm_ref, h_ref, g_ref, b_ref,
              out_ref, h2_scr, sum_scr, ssq_scr):
    ph = pl.program_id(0)
    i = pl.program_id(1)

    @pl.when(ph == 0)
    def _stats():
        num = jnp.concatenate([a0_ref[:, :H], a1_ref[:, :H]], axis=1)
        den = jnp.concatenate([a0_ref[:, H:], a1_ref[:, H:]], axis=1)
        h2 = (ah_ref[...] + num / (den + 1e-6)) * norm_ref[...]
        h2_scr[pl.ds(i * BN_BLK, BN_BLK), :] = h2

        @pl.when(i == 0)
        def _init():
            sum_scr[...] = jnp.zeros_like(sum_scr)
            ssq_scr[...] = jnp.zeros_like(ssq_scr)

        sum_scr[...] += jnp.sum(h2, axis=0, keepdims=True)
        ssq_scr[...] += jnp.sum(h2 * h2, axis=0, keepdims=True)

    @pl.when(ph == 1)
    def _apply():
        mean = sum_scr[...] / N
        var = ssq_scr[...] / N - mean * mean
        inv = lax.rsqrt(var + 1e-5)
        h2 = h2_scr[pl.ds(i * BN_BLK, BN_BLK), :]
        out_ref[...] = (h_ref[...] + (h2 - mean) * inv * g_ref[...]
                        + b_ref[...])


def _finalize(ah, acc0, acc1, norm, h, gamma, beta):
    return pl.pallas_call(
        _fin_body,
        grid=(2, BN_NB),
        in_specs=[
            pl.BlockSpec((BN_BLK, D), lambda p, i: (i, 0)),
            pl.BlockSpec((BN_BLK, D), lambda p, i: (i, 0)),
            pl.BlockSpec((BN_BLK, D), lambda p, i: (i, 0)),
            pl.BlockSpec((BN_BLK, 1), lambda p, i: (i, 0)),
            pl.BlockSpec((BN_BLK, D), lambda p, i: (i, 0)),
            pl.BlockSpec((1, D), lambda p, i: (0, 0)),
            pl.BlockSpec((1, D), lambda p, i: (0, 0)),
        ],
        out_specs=pl.BlockSpec((BN_BLK, D), lambda p, i: (i, 0)),
        out_shape=jax.ShapeDtypeStruct((N, D), jnp.float32),
        scratch_shapes=[
            pltpu.VMEM((N, D), jnp.float32),
            pltpu.VMEM((1, D), jnp.float32),
            pltpu.VMEM((1, D), jnp.float32),
        ],
    )(ah, acc0, acc1, norm, h, gamma, beta)


def kernel(h, edge_index, e, norm, WA, bA, WB, bB, WD, bD, WE, bE, gamma, beta):
    wcat = jnp.concatenate([WA, WB, WD, WE], axis=1)
    bcat = jnp.concatenate([bA, bB, bD, bE])[None, :]
    ah, bd, eh = _matmuls(h, norm, wcat, bcat)
    bd = bd.reshape(2 * N, D)
    eh = eh.reshape(2 * N, D)
    src = edge_index[0]
    dst = edge_index[1]
    acc = _edge_phase(bd, eh, src, dst)
    out = _finalize(ah, acc[:N], acc[NP:NP + N], norm, h,
                    gamma[None, :], beta[None, :])
    return (out, e)


# speedup vs baseline: 1.0202x; 1.0142x over previous
import jax
import jax.numpy as jnp
from jax import lax
from jax.experimental import pallas as pl
from jax.experimental.pallas import tpu as pltpu
from jax.experimental.pallas import tpu_sc as plsc

N = 10000
E = 320000
D = 128
H = D // 2

NS = 16
EPT = E // NS
CHUNK = 80
NCHUNK = EPT // CHUNK
ROWS_PT = 632


def _mm_body(h_ref, norm_ref, w_ref, b_ref, ah_ref, bd_ref, eh_ref):
    hh = h_ref[...] * norm_ref[...]
    p = jnp.dot(hh, w_ref[...], preferred_element_type=jnp.float32) + b_ref[...]
    ah_ref[...] = p[:, 0:128]
    b_part = p[:, 128:256]
    d_part = jnp.exp(-p[:, 256:384])
    e_part = jnp.exp(-p[:, 384:512])
    bd_ref[0] = jnp.concatenate([b_part[:, :H], d_part[:, :H]], axis=1)
    bd_ref[1] = jnp.concatenate([b_part[:, H:], d_part[:, H:]], axis=1)
    eh_ref[0] = e_part
    eh_ref[1] = jnp.concatenate([e_part[:, H:], e_part[:, :H]], axis=1)


def _matmuls(h, norm, wcat, bcat):
    bn = 1000
    nb = N // bn
    return pl.pallas_call(
        _mm_body,
        grid=(nb,),
        in_specs=[
            pl.BlockSpec((bn, D), lambda i: (i, 0)),
            pl.BlockSpec((bn, 1), lambda i: (i, 0)),
            pl.BlockSpec((D, 4 * D), lambda i: (0, 0)),
            pl.BlockSpec((1, 4 * D), lambda i: (0, 0)),
        ],
        out_specs=[
            pl.BlockSpec((bn, D), lambda i: (i, 0)),
            pl.BlockSpec((2, bn, D), lambda i: (0, i, 0)),
            pl.BlockSpec((2, bn, D), lambda i: (0, i, 0)),
        ],
        out_shape=[
            jax.ShapeDtypeStruct((N, D), jnp.float32),
            jax.ShapeDtypeStruct((2, N, D), jnp.float32),
            jax.ShapeDtypeStruct((2, N, D), jnp.float32),
        ],
    )(h, norm, wcat, bcat)


IDXB = 2000
CPB = IDXB // CHUNK


def _edge_body(bd_hbm, eh_hbm, src_hbm, dst_hbm, out0_hbm, out1_hbm,
               big_src, big_dst, idx_srcc, idx_dstc, idx_dsc, bd_v, eh_v,
               acc, sem0, sem1):
    c = lax.axis_index("c")
    s = lax.axis_index("s")
    c_n = c * N
    sems = (sem0, sem1)

    row0 = s * ROWS_PT
    zero16 = jnp.zeros((16,), jnp.float32)

    def zrow(i, carry0):
        for j in range(D // 16):
            bd_v.at[0][i, pl.ds(j * 16, 16)] = zero16
        return carry0

    lax.fori_loop(0, CHUNK, zrow, 0)

    def fill_rows(nrows):
        for m in range(nrows // CHUNK):
            pltpu.sync_copy(bd_v.at[0],
                            acc.at[pl.ds(row0 + m * CHUNK, CHUNK)])
        rem = nrows % CHUNK
        if rem:
            pltpu.sync_copy(bd_v.at[0].at[pl.ds(0, rem)],
                            acc.at[pl.ds(row0 + nrows - rem, rem)])

    @pl.when(s < NS - 1)
    def _zmain():
        fill_rows(ROWS_PT)

    @pl.when(s == NS - 1)
    def _ztail():
        fill_rows(N - (NS - 1) * ROWS_PT)

    plsc.subcore_barrier()

    base = s * EPT

    def load_batch(g):
        off = base + g * IDXB
        pltpu.sync_copy(src_hbm.at[pl.ds(off, IDXB)], big_src)
        pltpu.sync_copy(dst_hbm.at[pl.ds(off, IDXB)], big_dst)

    def prep_idx(b, k):
        lo = lax.rem(k, CPB) * CHUNK
        for j in range(CHUNK // 16):
            sl_in = pl.ds(lo + j * 16, 16)
            sl = pl.ds(j * 16, 16)
            idx_srcc.at[b][sl] = big_src[sl_in] + c_n
            dv = big_dst[sl_in]
            idx_dstc.at[b][sl] = dv + c_n
            idx_dsc.at[b][sl] = dv

    def start_gathers(b):
        pltpu.async_copy(bd_hbm.at[idx_srcc.at[b]], bd_v.at[b], sems[b])
        pltpu.async_copy(eh_hbm.at[idx_dstc.at[b]], eh_v.at[b], sems[b])

    def wait_gathers(b):
        pltpu.make_async_copy(bd_hbm.at[idx_srcc.at[b]], bd_v.at[b],
                              sems[b]).wait()
        pltpu.make_async_copy(eh_hbm.at[idx_dstc.at[b]], eh_v.at[b],
                              sems[b]).wait()

    load_batch(0)
    for b in range(2):
        prep_idx(b, b)
        start_gathers(b)

    def pair(p, carry):
        for b in range(2):
            k = 2 * p + b
            wait_gathers(b)

            def edge(i, carry2):
                for j in range(H // 16):
                    slj = pl.ds(j * 16, 16)
                    slj2 = pl.ds(H + j * 16, 16)
                    bh = bd_v.at[b][i, slj]
                    ed = bd_v.at[b][i, slj2]
                    ee = eh_v.at[b][i, slj]
                    sg = 1.0 / (1.0 + ed * ee)
                    bd_v.at[b][i, slj] = sg * bh
                    bd_v.at[b][i, slj2] = sg
                return carry2

            lax.fori_loop(0, CHUNK, edge, 0)
            pltpu.sync_copy(bd_v.at[b], acc.at[idx_dsc.at[b]], add=True)

            @pl.when(k < NCHUNK - 2)
            def _ahead():
                @pl.when(lax.rem(k + 2, CPB) == 0)
                def _refill():
                    load_batch((k + 2) // CPB)

                prep_idx(b, k + 2)
                start_gathers(b)

        return carry

    lax.fori_loop(0, NCHUNK // 2, pair, 0)
    plsc.subcore_barrier()

    def writeout(dst_hbm2):
        @pl.when(s < NS - 1)
        def _wmain():
            pltpu.sync_copy(acc.at[pl.ds(row0, ROWS_PT)],
                            dst_hbm2.at[pl.ds(row0, ROWS_PT)])

        @pl.when(s == NS - 1)
        def _wtail():
            tail = N - (NS - 1) * ROWS_PT
            pltpu.sync_copy(acc.at[pl.ds(row0, tail)],
                            dst_hbm2.at[pl.ds(row0, tail)])

    @pl.when(c == 0)
    def _w0():
        writeout(out0_hbm)

    @pl.when(c == 1)
    def _w1():
        writeout(out1_hbm)


def _edge_phase(bd, eh, src, dst):
    mesh = plsc.VectorSubcoreMesh(core_axis_name="c", subcore_axis_name="s")
    k = pl.kernel(
        _edge_body,
        out_type=[jax.ShapeDtypeStruct((N, D), jnp.float32),
                  jax.ShapeDtypeStruct((N, D), jnp.float32)],
        mesh=mesh,
        scratch_types=[
            pltpu.VMEM((IDXB,), jnp.int32),
            pltpu.VMEM((IDXB,), jnp.int32),
            pltpu.VMEM((2, CHUNK), jnp.int32),
            pltpu.VMEM((2, CHUNK), jnp.int32),
            pltpu.VMEM((2, CHUNK), jnp.int32),
            pltpu.VMEM((2, CHUNK, D), jnp.float32),
            pltpu.VMEM((2, CHUNK, D), jnp.float32),
            pltpu.VMEM_SHARED((N, D), jnp.float32),
            pltpu.SemaphoreType.DMA,
            pltpu.SemaphoreType.DMA,
        ],
    )
    return k(bd, eh, src, dst)


BN_BLK = 1000
BN_NB = N // BN_BLK


def _fin_body(ah_ref, a0_ref, a1_ref, norm_ref, h_ref, g_ref, b_ref,
              out_ref, h2_scr, sum_scr, ssq_scr):
    ph = pl.program_id(0)
    i = pl.program_id(1)

    @pl.when(ph == 0)
    def _stats():
        num = jnp.concatenate([a0_ref[:, :H], a1_ref[:, :H]], axis=1)
        den = jnp.concatenate([a0_ref[:, H:], a1_ref[:, H:]], axis=1)
        h2 = (ah_ref[...] + num / (den + 1e-6)) * norm_ref[...]
        h2_scr[pl.ds(i * BN_BLK, BN_BLK), :] = h2

        @pl.when(i == 0)
        def _init():
            sum_scr[...] = jnp.zeros_like(sum_scr)
            ssq_scr[...] = jnp.zeros_like(ssq_scr)

        sum_scr[...] += jnp.sum(h2, axis=0, keepdims=True)
        ssq_scr[...] += jnp.sum(h2 * h2, axis=0, keepdims=True)

    @pl.when(ph == 1)
    def _apply():
        mean = sum_scr[...] / N
        var = ssq_scr[...] / N - mean * mean
        inv = lax.rsqrt(var + 1e-5)
        h2 = h2_scr[pl.ds(i * BN_BLK, BN_BLK), :]
        out_ref[...] = (h_ref[...] + (h2 - mean) * inv * g_ref[...]
                        + b_ref[...])


def _finalize(ah, acc0, acc1, norm, h, gamma, beta):
    return pl.pallas_call(
        _fin_body,
        grid=(2, BN_NB),
        in_specs=[
            pl.BlockSpec((BN_BLK, D), lambda p, i: (i, 0)),
            pl.BlockSpec((BN_BLK, D), lambda p, i: (i, 0)),
            pl.BlockSpec((BN_BLK, D), lambda p, i: (i, 0)),
            pl.BlockSpec((BN_BLK, 1), lambda p, i: (i, 0)),
            pl.BlockSpec((BN_BLK, D), lambda p, i: (i, 0)),
            pl.BlockSpec((1, D), lambda p, i: (0, 0)),
            pl.BlockSpec((1, D), lambda p, i: (0, 0)),
        ],
        out_specs=pl.BlockSpec((BN_BLK, D), lambda p, i: (i, 0)),
        out_shape=jax.ShapeDtypeStruct((N, D), jnp.float32),
        scratch_shapes=[
            pltpu.VMEM((N, D), jnp.float32),
            pltpu.VMEM((1, D), jnp.float32),
            pltpu.VMEM((1, D), jnp.float32),
        ],
    )(ah, acc0, acc1, norm, h, gamma, beta)


def kernel(h, edge_index, e, norm, WA, bA, WB, bB, WD, bD, WE, bE, gamma, beta):
    wcat = jnp.concatenate([WA, WB, WD, WE], axis=1)
    bcat = jnp.concatenate([bA, bB, bD, bE])[None, :]
    ah, bd, eh = _matmuls(h, norm, wcat, bcat)
    bd = bd.reshape(2 * N, D)
    eh = eh.reshape(2 * N, D)
    src = edge_index[0]
    dst = edge_index[1]
    acc0, acc1 = _edge_phase(bd, eh, src, dst)
    out = _finalize(ah, acc0, acc1, norm, h, gamma[None, :], beta[None, :])
    return (out, e)
